# Initial kernel scaffold; baseline (speedup 1.0000x reference)
import jax
import jax.numpy as jnp
from jax.experimental import pallas as pl


def kernel(index, table, index_map):
    raise NotImplementedError("write your pallas kernel here")


# speedup vs baseline: 1.4134x; 1.4134x over previous
import functools

import jax
import jax.numpy as jnp
from jax import lax
from jax.experimental import pallas as pl
from jax.experimental.pallas import tpu as pltpu
from jax.experimental.pallas import tpu_sc as plsc

_VOCAB = 1_000_000
_DIM = 64
_NC, _NS, _L = 2, 16, 16
_NW = _NC * _NS
_C = 128
_NBUF = 8


def _body(idx_hbm, map_hbm, table_hbm, out_hbm, idx_v, row_v, rbuf,
          msem, gsem, wsem, *, nch):
    wid = lax.axis_index("s") * _NC + lax.axis_index("c")

    pltpu.sync_copy(idx_hbm.at[wid], idx_v)

    def remap_fire(j, carry):
        for k in range(_C // _L):
            v = idx_v[j, pl.ds(k * _L, _L)]
            idx_v[j, pl.ds(k * _L, _L)] = jnp.minimum(
                jnp.maximum(v, 0), _VOCAB)
        pltpu.make_async_copy(
            map_hbm.at[idx_v.at[j]], row_v.at[j], msem).start()
        return carry

    lax.fori_loop(0, nch, remap_fire, 0)

    def remap_drain(j, carry):
        pltpu.make_async_copy(
            map_hbm.at[idx_v.at[j]], row_v.at[j], msem).wait()
        return carry

    lax.fori_loop(0, nch, remap_drain, 0)

    out_base = wid * (nch * _C)

    def g_copy(j, b):
        return pltpu.make_async_copy(
            table_hbm.at[row_v.at[j]], rbuf.at[b], gsem.at[b])

    def w_copy(j, b):
        return pltpu.make_async_copy(
            rbuf.at[b], out_hbm.at[pl.ds(out_base + j * _C, _C)], wsem.at[b])

    for b in range(_NBUF):
        g_copy(b, b).start()

    def main(g, carry):
        j0 = g * _NBUF
        for b in range(_NBUF):
            j = j0 + b
            g_copy(j, b).wait()
            w_copy(j, b).start()
            nxt = j + _NBUF

            @pl.when(nxt < nch)
            def _refill():
                w_copy(j, b).wait()
                g_copy(nxt, b).start()

        return carry

    lax.fori_loop(0, nch // _NBUF, main, 0)

    for b in range(_NBUF):
        w_copy(nch - _NBUF + b, b).wait()


@functools.partial(jax.jit, static_argnames=("nch",))
def _lookup(idx, index_map, table, *, nch):
    n = _NW * nch * _C
    body = functools.partial(_body, nch=nch)
    return pl.kernel(
        body,
        out_type=jax.ShapeDtypeStruct((n, _DIM), jnp.float32),
        mesh=plsc.VectorSubcoreMesh(
            core_axis_name="c", subcore_axis_name="s",
            num_cores=_NC, num_subcores=_NS),
        scratch_types=[
            pltpu.VMEM((nch, _C), jnp.int32),
            pltpu.VMEM((nch, _C), jnp.int32),
            pltpu.VMEM((_NBUF, _C, _DIM), jnp.float32),
            pltpu.SemaphoreType.DMA,
            pltpu.SemaphoreType.DMA((_NBUF,)),
            pltpu.SemaphoreType.DMA((_NBUF,)),
        ],
        compiler_params=pltpu.CompilerParams(use_tc_tiling_on_sc=False),
    )(idx, index_map, table)


def kernel(index, table, index_map):
    b, f = index.shape
    n = b * f
    nch = n // (_NW * _C)
    assert nch * _NW * _C == n and nch % _NBUF == 0
    idx = index.reshape(_NW, nch, _C)
    out = _lookup(idx, index_map, table, nch=nch)
    return out.reshape(b, f, _DIM)
